# baseline (speedup 1.0000x reference)
import functools
import math

import jax
import jax.numpy as jnp
from jax import lax
from jax.experimental import pallas as pl
from jax.experimental.pallas import tpu as pltpu
from jax.experimental.pallas import tpu_sc as plsc

VOCAB = 1000000
EMB = 64
SCALE = math.sqrt(EMB)

NC = 2
NS = 16
NW = NC * NS

SPLITS = (0, 128)
SIZES = (128, 72)


def _sc_embed(B, T):
    assert B % NW == 0
    rows_per_w = B // NW
    tok_per_w = rows_per_w * T
    mesh = plsc.VectorSubcoreMesh(core_axis_name="c", subcore_axis_name="s")

    @functools.partial(
        pl.kernel,
        mesh=mesh,
        out_type=jax.ShapeDtypeStruct((B, T, EMB), jnp.float32),
        scratch_types=[
            pltpu.VMEM((tok_per_w,), jnp.int32),
            pltpu.VMEM((2, T, EMB), jnp.float32),
            pltpu.SemaphoreType.DMA,
        ],
        compiler_params=pltpu.CompilerParams(use_tc_tiling_on_sc=False),
    )
    def k(tokens_hbm, table_hbm, out_hbm, idx_all, rows_v, gsem):
        wid = lax.axis_index("s") * NC + lax.axis_index("c")
        base = wid * tok_per_w

        pltpu.sync_copy(tokens_hbm.at[pl.ds(base, tok_per_w)], idx_all)

        def start_gathers(g, p):
            for off, sz in zip(SPLITS, SIZES):
                pltpu.async_copy(
                    table_hbm.at[idx_all.at[pl.ds(g * T + off, sz)]],
                    rows_v.at[p, pl.ds(off, sz)],
                    gsem,
                )

        start_gathers(0, 0)

        def chunk_body(g, carry):
            p = lax.rem(g, 2)

            @pl.when(g + 1 < rows_per_w)
            def _():
                start_gathers(g + 1, 1 - p)

            pltpu.make_async_copy(
                table_hbm.at[pl.ds(0, T)], rows_v.at[p], gsem
            ).wait()

            pltpu.sync_copy(rows_v.at[p], out_hbm.at[wid * rows_per_w + g])
            return carry

        lax.fori_loop(0, rows_per_w, chunk_body, 0)

    return k


def kernel(tokens, table):
    b, t = tokens.shape
    flat = tokens.reshape(b * t).astype(jnp.int32)
    return _sc_embed(b, t)(flat, table) * SCALE


# speedup vs baseline: 1.0900x; 1.0900x over previous
import functools
import math

import jax
import jax.numpy as jnp
from jax import lax
from jax.experimental import pallas as pl
from jax.experimental.pallas import tpu as pltpu
from jax.experimental.pallas import tpu_sc as plsc

VOCAB = 1000000
EMB = 64
PADW = 128
SCALE = math.sqrt(EMB)

NC = 2
NS = 16
NW = NC * NS

SPLITS = (0, 128)
SIZES = (128, 72)


def _sc_embed(B, T):
    assert B % NW == 0
    rows_per_w = B // NW
    tok_per_w = rows_per_w * T
    mesh = plsc.VectorSubcoreMesh(core_axis_name="c", subcore_axis_name="s")

    @functools.partial(
        pl.kernel,
        mesh=mesh,
        out_type=jax.ShapeDtypeStruct((B, T, PADW), jnp.float32),
        scratch_types=[
            pltpu.VMEM((tok_per_w,), jnp.int32),
            pltpu.VMEM((2, T, EMB), jnp.float32),
            pltpu.VMEM((2, T, PADW), jnp.float32),
            pltpu.SemaphoreType.DMA,
        ],
        compiler_params=pltpu.CompilerParams(use_tc_tiling_on_sc=False),
    )
    def k(tokens_hbm, table_hbm, out_hbm, idx_all, rows_c, rows_p, gsem):
        wid = lax.axis_index("s") * NC + lax.axis_index("c")
        base = wid * tok_per_w

        pltpu.sync_copy(tokens_hbm.at[pl.ds(base, tok_per_w)], idx_all)

        def start_gathers(g, p):
            for off, sz in zip(SPLITS, SIZES):
                pltpu.async_copy(
                    table_hbm.at[idx_all.at[pl.ds(g * T + off, sz)]],
                    rows_c.at[p, pl.ds(off, sz)],
                    gsem,
                )

        start_gathers(0, 0)

        def chunk_body(g, carry):
            p = lax.rem(g, 2)

            @pl.when(g + 1 < rows_per_w)
            def _():
                start_gathers(g + 1, 1 - p)

            pltpu.make_async_copy(
                table_hbm.at[pl.ds(0, T)], rows_c.at[p], gsem
            ).wait()

            def scale_body(i, c2):
                for j in range(EMB // 16):
                    sl = pl.ds(j * 16, 16)
                    rows_p[p, i, sl] = rows_c[p, i, sl] * SCALE
                return c2

            lax.fori_loop(0, T, scale_body, 0, unroll=4)

            pltpu.sync_copy(rows_p.at[p], out_hbm.at[wid * rows_per_w + g])
            return carry

        lax.fori_loop(0, rows_per_w, chunk_body, 0)

    return k


def kernel(tokens, table):
    b, t = tokens.shape
    flat = tokens.reshape(b * t).astype(jnp.int32)
    wide = _sc_embed(b, t)(flat, table)
    return wide[:, :, :EMB]


# speedup vs baseline: 1.1918x; 1.0934x over previous
import functools
import math

import jax
import jax.numpy as jnp
from jax import lax
from jax.experimental import pallas as pl
from jax.experimental.pallas import tpu as pltpu
from jax.experimental.pallas import tpu_sc as plsc

VOCAB = 1000000
EMB = 64
PADW = 128
SCALE = math.sqrt(EMB)

NC = 2
NS = 16
NW = NC * NS

SPLITS = (0, 128)
SIZES = (128, 72)


def _sc_embed(B, T):
    assert B % NW == 0
    rows_per_w = B // NW
    tok_per_w = rows_per_w * T
    mesh = plsc.VectorSubcoreMesh(core_axis_name="c", subcore_axis_name="s")

    @functools.partial(
        pl.kernel,
        mesh=mesh,
        out_type=jax.ShapeDtypeStruct((B, T, PADW), jnp.float32),
        scratch_types=[
            pltpu.VMEM((tok_per_w,), jnp.int32),
            pltpu.VMEM((2, T, EMB), jnp.float32),
            pltpu.VMEM((2, T, PADW), jnp.float32),
            pltpu.SemaphoreType.DMA,
            pltpu.SemaphoreType.DMA,
        ],
        compiler_params=pltpu.CompilerParams(use_tc_tiling_on_sc=False),
    )
    def k(tokens_hbm, table_hbm, out_hbm, idx_all, rows_c, rows_p, gsem, osem):
        wid = lax.axis_index("s") * NC + lax.axis_index("c")
        base = wid * tok_per_w

        pltpu.sync_copy(tokens_hbm.at[pl.ds(base, tok_per_w)], idx_all)

        def start_gathers(g, p):
            for off, sz in zip(SPLITS, SIZES):
                pltpu.async_copy(
                    table_hbm.at[idx_all.at[pl.ds(g * T + off, sz)]],
                    rows_c.at[p, pl.ds(off, sz)],
                    gsem,
                )

        start_gathers(0, 0)

        def chunk_body(g, carry):
            p = lax.rem(g, 2)

            @pl.when(g + 1 < rows_per_w)
            def _():
                start_gathers(g + 1, 1 - p)

            pltpu.make_async_copy(
                table_hbm.at[pl.ds(0, T)], rows_c.at[p], gsem
            ).wait()

            @pl.when(g >= 2)
            def _():
                pltpu.make_async_copy(
                    rows_p.at[p], out_hbm.at[0], osem
                ).wait()

            def scale_body(i, c2):
                for j in range(EMB // 16):
                    sl = pl.ds(j * 16, 16)
                    rows_p[p, i, sl] = rows_c[p, i, sl] * SCALE
                return c2

            lax.fori_loop(0, T, scale_body, 0, unroll=4)

            pltpu.async_copy(
                rows_p.at[p], out_hbm.at[wid * rows_per_w + g], osem
            )
            return carry

        lax.fori_loop(0, rows_per_w, chunk_body, 0)

        for _ in range(2):
            pltpu.make_async_copy(rows_p.at[0], out_hbm.at[0], osem).wait()

    return k


def kernel(tokens, table):
    b, t = tokens.shape
    flat = tokens.reshape(b * t).astype(jnp.int32)
    wide = _sc_embed(b, t)(flat, table)
    return wide[:, :, :EMB]


# speedup vs baseline: 1.5870x; 1.3316x over previous
import functools
import math

import jax
import jax.numpy as jnp
from jax import lax
from jax.experimental import pallas as pl
from jax.experimental.pallas import tpu as pltpu
from jax.experimental.pallas import tpu_sc as plsc

VOCAB = 1000000
EMB = 64
PADW = 128
SCALE = math.sqrt(EMB)

NC = 2
NS = 16
NW = NC * NS

SPLITS = (0, 128)
SIZES = (128, 72)


def _sc_embed(B, T):
    assert B % NW == 0
    rows_per_w = B // NW
    tok_per_w = rows_per_w * T
    mesh = plsc.VectorSubcoreMesh(core_axis_name="c", subcore_axis_name="s")

    @functools.partial(
        pl.kernel,
        mesh=mesh,
        out_type=jax.ShapeDtypeStruct((B, T, PADW), jnp.float32),
        scratch_types=[
            pltpu.VMEM((tok_per_w,), jnp.int32),
            pltpu.VMEM((2, T, EMB), jnp.float32),
            pltpu.SemaphoreType.DMA,
            pltpu.SemaphoreType.DMA,
        ],
        compiler_params=pltpu.CompilerParams(use_tc_tiling_on_sc=False),
    )
    def k(tokens_hbm, table_hbm, out_hbm, idx_all, rows_c, gsem, osem):
        wid = lax.axis_index("s") * NC + lax.axis_index("c")
        base = wid * tok_per_w

        pltpu.sync_copy(tokens_hbm.at[pl.ds(base, tok_per_w)], idx_all)

        def start_gathers(g, p):
            for off, sz in zip(SPLITS, SIZES):
                pltpu.async_copy(
                    table_hbm.at[idx_all.at[pl.ds(g * T + off, sz)]],
                    rows_c.at[p, pl.ds(off, sz)],
                    gsem,
                )

        start_gathers(0, 0)

        def chunk_body(g, carry):
            p = lax.rem(g, 2)

            @pl.when(g + 1 < rows_per_w)
            def _():
                @pl.when(g >= 1)
                def _():
                    pltpu.make_async_copy(
                        rows_c.at[1 - p],
                        out_hbm.at[0, pl.ds(0, T), pl.ds(0, EMB)],
                        osem,
                    ).wait()

                start_gathers(g + 1, 1 - p)

            pltpu.make_async_copy(
                table_hbm.at[pl.ds(0, T)], rows_c.at[p], gsem
            ).wait()

            def scale_body(i, c2):
                for j in range(EMB // 16):
                    sl = pl.ds(j * 16, 16)
                    rows_c[p, i, sl] = rows_c[p, i, sl] * SCALE
                return c2

            lax.fori_loop(0, T, scale_body, 0, unroll=4)

            pltpu.async_copy(
                rows_c.at[p],
                out_hbm.at[wid * rows_per_w + g, pl.ds(0, T), pl.ds(0, EMB)],
                osem,
            )
            return carry

        lax.fori_loop(0, rows_per_w, chunk_body, 0)

        for _ in range(2):
            pltpu.make_async_copy(
                rows_c.at[0], out_hbm.at[0, pl.ds(0, T), pl.ds(0, EMB)], osem
            ).wait()

    return k


def kernel(tokens, table):
    b, t = tokens.shape
    flat = tokens.reshape(b * t).astype(jnp.int32)
    wide = _sc_embed(b, t)(flat, table)
    return wide[:, :, :EMB]
